# baseline (device time: 12375 ns/iter reference)
import jax
import jax.numpy as jnp
from jax import lax
from jax.experimental import pallas as pl
from jax.experimental.pallas import tpu as pltpu

_RM = 256


def kernel(x, dy, gamma):
    m, d = x.shape

    def body(
        x_hbm, dy_hbm, out_ref,
        xb, dyb, colp, planep, local_sems,
        sendz, recvz, sendxy, recvxy,
    ):
        my_x = lax.axis_index("x")
        my_y = lax.axis_index("y")
        my_z = lax.axis_index("z")
        pr = my_x * 2 + my_y
        rank = my_x * 4 + my_z
        r0 = rank * _RM

        z_peer = [(my_x, my_y, (my_z + p) % 4) for p in range(1, 4)]
        xy_peer = [
            (((pr + p) % 4) // 2, ((pr + p) % 4) % 2, my_z) for p in range(1, 4)
        ]

        barrier = pltpu.get_barrier_semaphore()
        for tgt in z_peer + xy_peer:
            pl.semaphore_signal(
                barrier, inc=1, device_id=tgt,
                device_id_type=pl.DeviceIdType.MESH,
            )

        cp_x = pltpu.make_async_copy(
            x_hbm.at[pl.ds(r0, _RM), :], xb, local_sems.at[0]
        )
        cp_dy = pltpu.make_async_copy(
            dy_hbm.at[pl.ds(r0, _RM), :], dyb, local_sems.at[1]
        )
        cp_x.start()
        cp_dy.start()

        cp_x.wait()
        xv = xb[:, :]
        mu = jnp.mean(xv, axis=1, keepdims=True)
        xc = xv - mu
        var = jnp.mean(xc * xc, axis=1, keepdims=True)
        xhat = xc * lax.rsqrt(var + 1e-5)

        cp_dy.wait()
        dyv = dyb[:, :]
        colp[my_z, 0, :] = jnp.sum(dyv * xhat, axis=0)
        colp[my_z, 1, :] = jnp.sum(dyv, axis=0)

        pl.semaphore_wait(barrier, 6)

        sends = []
        for p in range(1, 4):
            rdma = pltpu.make_async_remote_copy(
                src_ref=colp.at[my_z],
                dst_ref=colp.at[my_z],
                send_sem=sendz.at[p],
                recv_sem=recvz.at[4 - p],
                device_id=z_peer[p - 1],
                device_id_type=pl.DeviceIdType.MESH,
            )
            rdma.start()
            sends.append(rdma)

        for q in range(1, 4):
            recv = pltpu.make_async_remote_copy(
                src_ref=colp.at[my_z],
                dst_ref=colp.at[(my_z + q) % 4],
                send_sem=sendz.at[q],
                recv_sem=recvz.at[q],
                device_id=(my_x, my_y, my_z),
                device_id_type=pl.DeviceIdType.MESH,
            )
            recv.wait_recv()

        planep[pr, :, :] = jnp.sum(colp[:, :, :], axis=0)

        for p in range(1, 4):
            rdma = pltpu.make_async_remote_copy(
                src_ref=planep.at[pr],
                dst_ref=planep.at[pr],
                send_sem=sendxy.at[p],
                recv_sem=recvxy.at[4 - p],
                device_id=xy_peer[p - 1],
                device_id_type=pl.DeviceIdType.MESH,
            )
            rdma.start()
            sends.append(rdma)

        for q in range(1, 4):
            recv = pltpu.make_async_remote_copy(
                src_ref=planep.at[pr],
                dst_ref=planep.at[(pr + q) % 4],
                send_sem=sendxy.at[q],
                recv_sem=recvxy.at[q],
                device_id=(my_x, my_y, my_z),
                device_id_type=pl.DeviceIdType.MESH,
            )
            recv.wait_recv()

        out_ref[:, :] = jnp.sum(planep[:, :, :], axis=0)

        for rdma in sends:
            rdma.wait_send()

    return pl.pallas_call(
        body,
        out_shape=jax.ShapeDtypeStruct((2, d), jnp.float32),
        in_specs=[
            pl.BlockSpec(memory_space=pltpu.MemorySpace.HBM),
            pl.BlockSpec(memory_space=pltpu.MemorySpace.HBM),
        ],
        out_specs=pl.BlockSpec(memory_space=pltpu.VMEM),
        scratch_shapes=[
            pltpu.VMEM((_RM, d), jnp.float32),
            pltpu.VMEM((_RM, d), jnp.float32),
            pltpu.VMEM((4, 2, d), jnp.float32),
            pltpu.VMEM((4, 2, d), jnp.float32),
            pltpu.SemaphoreType.DMA((2,)),
            pltpu.SemaphoreType.DMA((4,)),
            pltpu.SemaphoreType.DMA((4,)),
            pltpu.SemaphoreType.DMA((4,)),
            pltpu.SemaphoreType.DMA((4,)),
        ],
        compiler_params=pltpu.CompilerParams(collective_id=0),
    )(
        pltpu.with_memory_space_constraint(x, pltpu.MemorySpace.HBM),
        pltpu.with_memory_space_constraint(dy, pltpu.MemorySpace.HBM),
    )
